# baseline (speedup 1.0000x reference)
import functools

import numpy as np

import jax
import jax.numpy as jnp
from jax import lax
from jax.experimental import pallas as pl
from jax.experimental.pallas import tpu as pltpu
from jax.experimental.pallas import tpu_sc as plsc

_N = 10000
_D = 128
_H = 64
_R = 7
_E = 320000

_NW = 32
_EPW = _E // _NW
_LANES = 16
_BINS = 4096
_BN = 1000
_NG = _N // _BN

_SEG = np.repeat(np.eye(8, dtype=np.float32), _BINS // 8, axis=1)


def _sc_hist(t_idx):
    mesh = plsc.VectorSubcoreMesh(core_axis_name="c", subcore_axis_name="s")

    @functools.partial(
        pl.kernel,
        mesh=mesh,
        out_type=jax.ShapeDtypeStruct((_NW * _BINS,), jnp.float32),
        scratch_types=[
            pltpu.VMEM((_EPW,), jnp.int32),
            pltpu.VMEM((_BINS * _LANES,), jnp.float32),
            pltpu.VMEM((_BINS,), jnp.float32),
        ],
        compiler_params=pltpu.CompilerParams(needs_layout_passes=False),
    )
    def hist(t_hbm, out_hbm, tv, hv, rv):
        wid = lax.axis_index("s") * 2 + lax.axis_index("c")
        pltpu.sync_copy(t_hbm.at[pl.ds(wid * _EPW, _EPW)], tv)

        zero16 = jnp.zeros((_LANES,), jnp.float32)

        def zbody(i, carry):
            for u in range(8):
                hv[pl.ds(i * 128 + u * 16, 16)] = zero16
            return carry

        lax.fori_loop(0, _BINS * _LANES // 128, zbody, 0)

        lanes = lax.iota(jnp.int32, _LANES)
        lane_off = lanes * _BINS
        ones = jnp.ones((_LANES,), jnp.float32)

        def body(i, carry):
            t = tv[pl.ds(i * _LANES, _LANES)]
            plsc.addupdate_scatter(hv, [t + lane_off], ones)
            return carry

        lax.fori_loop(0, _EPW // _LANES, body, 0)

        def rbody(v, carry):
            acc = hv[pl.ds(v * 16, 16)]
            for l in range(1, _LANES):
                acc = acc + hv[pl.ds(l * _BINS + v * 16, 16)]
            rv[pl.ds(v * 16, 16)] = acc
            return carry

        lax.fori_loop(0, _BINS // 16, rbody, 0)
        pltpu.sync_copy(rv, out_hbm.at[pl.ds(wid * _BINS, _BINS)])

    return hist(t_idx)


def _enc_body(x_ref, w1t_ref, b1_ref, g1_ref, beta1_ref, w2t_ref, b2_ref,
              gn_ref, bn_ref, nodes8_ref, sum_ref, max_ref):
    g = pl.program_id(0)
    x = x_ref[...]
    h = jnp.dot(x, w1t_ref[...], preferred_element_type=jnp.float32)
    h = h + b1_ref[...]
    mu = jnp.mean(h, axis=-1, keepdims=True)
    var = jnp.mean((h - mu) ** 2, axis=-1, keepdims=True)
    h = (h - mu) / jnp.sqrt(var + 1e-5) * g1_ref[...] + beta1_ref[...]
    h = jnp.maximum(h, 0.0)
    n = jnp.dot(h, w2t_ref[...], preferred_element_type=jnp.float32)
    n = jnp.maximum(n + b2_ref[...], 0.0)

    @pl.when(g == 0)
    def _():
        nodes8_ref[...] = n[:8]
        sum_ref[...] = jnp.zeros((1, _H), jnp.float32)
        max_ref[...] = jnp.full((1, _H), -jnp.inf, jnp.float32)

    mu2 = jnp.mean(n, axis=-1, keepdims=True)
    var2 = jnp.mean((n - mu2) ** 2, axis=-1, keepdims=True)
    ln = (n - mu2) / jnp.sqrt(var2 + 1e-5) * gn_ref[...] + bn_ref[...]
    rows = lax.broadcasted_iota(jnp.int32, (_BN, 1), 0) + g * _BN
    keep = rows >= 8
    sum_ref[...] += jnp.sum(jnp.where(keep, ln, 0.0), axis=0, keepdims=True)
    max_ref[...] = jnp.maximum(
        max_ref[...],
        jnp.max(jnp.where(keep, ln, -jnp.inf), axis=0, keepdims=True))


def _encode_pool(block_features, w1t, b1, g1, beta1, w2t, b2, gn, bn):
    full = lambda s: pl.BlockSpec(s, lambda i: (0, 0))
    return pl.pallas_call(
        _enc_body,
        grid=(_NG,),
        in_specs=[
            pl.BlockSpec((_BN, _D), lambda i: (i, 0)),
            full((_D, _H)), full((1, _H)), full((1, _H)), full((1, _H)),
            full((_H, _H)), full((1, _H)), full((1, _H)), full((1, _H)),
        ],
        out_specs=[full((8, _H)), full((1, _H)), full((1, _H))],
        out_shape=[
            jax.ShapeDtypeStruct((8, _H), jnp.float32),
            jax.ShapeDtypeStruct((1, _H), jnp.float32),
            jax.ShapeDtypeStruct((1, _H), jnp.float32),
        ],
    )(block_features, w1t, b1, g1, beta1, w2t, b2, gn, bn)


def _fin_body(hists_ref, nodes8_ref, rel_ref, sumin_ref, maxin_ref, seg_ref,
              wms_ref, wmd_ref, wmr_ref, c0_ref, bm1_ref, wm2t_ref, bm2_ref,
              gn_ref, bn_ref, wg1t_ref, bg1_ref, wg2t_ref, bg2_ref, out_ref):
    counts = jnp.sum(hists_ref[...].reshape(_NW, _BINS), axis=0)
    n8 = nodes8_ref[...]
    rel8 = jnp.concatenate(
        [rel_ref[...], jnp.zeros((1, _H), jnp.float32)], axis=0)
    pre_d = jnp.dot(n8, wmd_ref[...], preferred_element_type=jnp.float32)
    pre_s = jnp.dot(n8, wms_ref[...], preferred_element_type=jnp.float32)
    pre_r = jnp.dot(rel8, wmr_ref[...], preferred_element_type=jnp.float32)
    wvals = lax.broadcasted_iota(jnp.int32, (8, 1), 0).astype(jnp.float32)
    pre_w = wvals * c0_ref[...]
    ds = (pre_d[:, None, :] + pre_s[None, :, :]).reshape(64, _H)
    rw = (pre_r[:, None, :] + pre_w[None, :, :]).reshape(64, _H)
    h1 = jnp.maximum(ds[:, None, :] + rw[None, :, :] + bm1_ref[...], 0.0)
    h1 = h1.reshape(_BINS, _H)
    msg = jnp.dot(h1, wm2t_ref[...], preferred_element_type=jnp.float32)
    msg = msg + bm2_ref[...]
    wmsg = msg * counts[:, None]
    agg8 = jnp.dot(seg_ref[...], wmsg,
                   preferred_element_type=jnp.float32)
    x8 = n8 + agg8
    mu = jnp.mean(x8, axis=-1, keepdims=True)
    var = jnp.mean((x8 - mu) ** 2, axis=-1, keepdims=True)
    ln8 = (x8 - mu) / jnp.sqrt(var + 1e-5) * gn_ref[...] + bn_ref[...]
    total = sumin_ref[...] + jnp.sum(ln8, axis=0, keepdims=True)
    mx = jnp.maximum(maxin_ref[...], jnp.max(ln8, axis=0, keepdims=True))
    graph = jnp.concatenate([total / float(_N), mx], axis=-1)
    z = jnp.dot(graph, wg1t_ref[...], preferred_element_type=jnp.float32)
    z = jnp.maximum(z + bg1_ref[...], 0.0)
    o = jnp.dot(z, wg2t_ref[...], preferred_element_type=jnp.float32)
    out_ref[...] = o + bg2_ref[...]


def _finalize(hists, nodes8, rel_emb, sum_in, max_in, seg, wms, wmd, wmr, c0,
              bm1, wm2t, bm2, gn, bn, wg1t, bg1, wg2t, bg2):
    return pl.pallas_call(
        _fin_body,
        out_shape=jax.ShapeDtypeStruct((1, _H), jnp.float32),
    )(hists, nodes8, rel_emb, sum_in, max_in, seg, wms, wmd, wmr, c0,
      bm1, wm2t, bm2, gn, bn, wg1t, bg1, wg2t, bg2)


def kernel(block_features, typed_edges, W1, b1, g1, beta1, W2, b2, rel_emb,
           Wm1, bm1, Wm2, bm2, gn, bn, Wg1, bg1, Wg2, bg2):
    te = typed_edges.astype(jnp.int32)
    tw = jnp.array([64, 512, 8, 1], dtype=jnp.int32)
    t_idx = lax.dot_general(te, tw, (((1,), (0,)), ((), ())),
                            preferred_element_type=jnp.int32)
    hists = _sc_hist(t_idx)

    r2 = lambda v: v.reshape(1, _H)
    nodes8, sum_in, max_in = _encode_pool(
        block_features, W1.T, r2(b1), r2(g1), r2(beta1), W2.T, r2(b2),
        r2(gn), r2(bn))

    wms = Wm1[:, :_H].T
    wmd = Wm1[:, _H:2 * _H].T
    wmr = Wm1[:, 2 * _H:3 * _H].T
    c0 = Wm1[:, 3 * _H].reshape(1, _H)

    out = _finalize(hists, nodes8, rel_emb, sum_in, max_in, jnp.asarray(_SEG),
                    wms, wmd, wmr, c0, r2(bm1), Wm2.T, r2(bm2), r2(gn),
                    r2(bn), Wg1.T, r2(bg1), Wg2.T, r2(bg2))
    return out.reshape(_H)


# speedup vs baseline: 1.2204x; 1.2204x over previous
import functools

import numpy as np

import jax
import jax.numpy as jnp
from jax import lax
from jax.experimental import pallas as pl
from jax.experimental.pallas import tpu as pltpu
from jax.experimental.pallas import tpu_sc as plsc

_N = 10000
_D = 128
_H = 64
_R = 7
_E = 320000

_NW = 32
_EPW = _E // _NW
_LANES = 16
_BINS = 4096
_BN = 1000
_NG = _N // _BN

_SEG = np.repeat(np.eye(8, dtype=np.float32), _BINS // 8, axis=1)


def _sc_hist(t_idx):
    mesh = plsc.VectorSubcoreMesh(core_axis_name="c", subcore_axis_name="s")

    @functools.partial(
        pl.kernel,
        mesh=mesh,
        out_type=jax.ShapeDtypeStruct((_NW * _BINS,), jnp.float32),
        scratch_types=[
            pltpu.VMEM((_EPW,), jnp.int32),
            pltpu.VMEM((_BINS * _LANES,), jnp.float32),
            pltpu.VMEM((_BINS,), jnp.float32),
        ],
        compiler_params=pltpu.CompilerParams(needs_layout_passes=False),
    )
    def hist(t_hbm, out_hbm, tv, hv, rv):
        wid = lax.axis_index("s") * 2 + lax.axis_index("c")
        pltpu.sync_copy(t_hbm.at[pl.ds(wid * _EPW, _EPW)], tv)

        zero16 = jnp.zeros((_LANES,), jnp.float32)

        def zbody(i, carry):
            for u in range(8):
                hv[pl.ds(i * 128 + u * 16, 16)] = zero16
            return carry

        lax.fori_loop(0, _BINS * _LANES // 128, zbody, 0)

        lanes = lax.iota(jnp.int32, _LANES)
        lane_off = lanes * _BINS
        ones = jnp.ones((_LANES,), jnp.float32)

        def body(i, carry):
            t = tv[pl.ds(i * _LANES, _LANES)]
            plsc.addupdate_scatter(hv, [t + lane_off], ones)
            return carry

        lax.fori_loop(0, _EPW // _LANES, body, 0)

        def rbody(v, carry):
            acc = hv[pl.ds(v * 16, 16)]
            for l in range(1, _LANES):
                acc = acc + hv[pl.ds(l * _BINS + v * 16, 16)]
            rv[pl.ds(v * 16, 16)] = acc
            return carry

        lax.fori_loop(0, _BINS // 16, rbody, 0)
        pltpu.sync_copy(rv, out_hbm.at[pl.ds(wid * _BINS, _BINS)])

    return hist(t_idx)


def _enc_body(x_ref, w1t_ref, b1_ref, g1_ref, beta1_ref, w2t_ref, b2_ref,
              gn_ref, bn_ref, nodes8_ref, sum_ref, max_ref):
    g = pl.program_id(0)
    x = x_ref[...]
    h = jnp.dot(x, w1t_ref[...], preferred_element_type=jnp.float32)
    h = h + b1_ref[...]
    mu = jnp.mean(h, axis=-1, keepdims=True)
    var = jnp.mean((h - mu) ** 2, axis=-1, keepdims=True)
    h = (h - mu) / jnp.sqrt(var + 1e-5) * g1_ref[...] + beta1_ref[...]
    h = jnp.maximum(h, 0.0)
    n = jnp.dot(h, w2t_ref[...], preferred_element_type=jnp.float32)
    n = jnp.maximum(n + b2_ref[...], 0.0)

    @pl.when(g == 0)
    def _():
        nodes8_ref[...] = n[:8]
        sum_ref[...] = jnp.zeros((1, _H), jnp.float32)
        max_ref[...] = jnp.full((1, _H), -jnp.inf, jnp.float32)

    mu2 = jnp.mean(n, axis=-1, keepdims=True)
    var2 = jnp.mean((n - mu2) ** 2, axis=-1, keepdims=True)
    ln = (n - mu2) / jnp.sqrt(var2 + 1e-5) * gn_ref[...] + bn_ref[...]
    rows = lax.broadcasted_iota(jnp.int32, (_BN, 1), 0) + g * _BN
    keep = rows >= 8
    sum_ref[...] += jnp.sum(jnp.where(keep, ln, 0.0), axis=0, keepdims=True)
    max_ref[...] = jnp.maximum(
        max_ref[...],
        jnp.max(jnp.where(keep, ln, -jnp.inf), axis=0, keepdims=True))


def _encode_pool(block_features, w1t, b1, g1, beta1, w2t, b2, gn, bn):
    full = lambda s: pl.BlockSpec(s, lambda i: (0, 0))
    return pl.pallas_call(
        _enc_body,
        grid=(_NG,),
        in_specs=[
            pl.BlockSpec((_BN, _D), lambda i: (i, 0)),
            full((_D, _H)), full((1, _H)), full((1, _H)), full((1, _H)),
            full((_H, _H)), full((1, _H)), full((1, _H)), full((1, _H)),
        ],
        out_specs=[full((8, _H)), full((1, _H)), full((1, _H))],
        out_shape=[
            jax.ShapeDtypeStruct((8, _H), jnp.float32),
            jax.ShapeDtypeStruct((1, _H), jnp.float32),
            jax.ShapeDtypeStruct((1, _H), jnp.float32),
        ],
    )(block_features, w1t, b1, g1, beta1, w2t, b2, gn, bn)


def _fin_body(hists_ref, nodes8_ref, rel_ref, sumin_ref, maxin_ref, seg_ref,
              wms_ref, wmd_ref, wmr_ref, c0_ref, bm1_ref, wm2t_ref, bm2_ref,
              gn_ref, bn_ref, wg1t_ref, bg1_ref, wg2t_ref, bg2_ref, out_ref):
    counts = jnp.sum(hists_ref[...].reshape(_NW, _BINS), axis=0)
    n8 = nodes8_ref[...]
    rel8 = jnp.concatenate(
        [rel_ref[...], jnp.zeros((1, _H), jnp.float32)], axis=0)
    pre_d = jnp.dot(n8, wmd_ref[...], preferred_element_type=jnp.float32)
    pre_s = jnp.dot(n8, wms_ref[...], preferred_element_type=jnp.float32)
    pre_r = jnp.dot(rel8, wmr_ref[...], preferred_element_type=jnp.float32)
    wvals = lax.broadcasted_iota(jnp.int32, (8, 1), 0).astype(jnp.float32)
    pre_w = wvals * c0_ref[...]
    ds = (pre_d[:, None, :] + pre_s[None, :, :]).reshape(64, _H)
    rw = (pre_r[:, None, :] + pre_w[None, :, :]).reshape(64, _H)
    h1 = jnp.maximum(ds[:, None, :] + rw[None, :, :] + bm1_ref[...], 0.0)
    h1 = h1.reshape(_BINS, _H)
    msg = jnp.dot(h1, wm2t_ref[...], preferred_element_type=jnp.float32)
    msg = msg + bm2_ref[...]
    wmsg = msg * counts[:, None]
    agg8 = jnp.dot(seg_ref[...], wmsg,
                   preferred_element_type=jnp.float32)
    x8 = n8 + agg8
    mu = jnp.mean(x8, axis=-1, keepdims=True)
    var = jnp.mean((x8 - mu) ** 2, axis=-1, keepdims=True)
    ln8 = (x8 - mu) / jnp.sqrt(var + 1e-5) * gn_ref[...] + bn_ref[...]
    total = sumin_ref[...] + jnp.sum(ln8, axis=0, keepdims=True)
    mx = jnp.maximum(maxin_ref[...], jnp.max(ln8, axis=0, keepdims=True))
    graph = jnp.concatenate([total / float(_N), mx], axis=-1)
    z = jnp.dot(graph, wg1t_ref[...], preferred_element_type=jnp.float32)
    z = jnp.maximum(z + bg1_ref[...], 0.0)
    o = jnp.dot(z, wg2t_ref[...], preferred_element_type=jnp.float32)
    out_ref[...] = o + bg2_ref[...]


def _finalize(hists, nodes8, rel_emb, sum_in, max_in, seg, wms, wmd, wmr, c0,
              bm1, wm2t, bm2, gn, bn, wg1t, bg1, wg2t, bg2):
    return pl.pallas_call(
        _fin_body,
        out_shape=jax.ShapeDtypeStruct((1, _H), jnp.float32),
    )(hists, nodes8, rel_emb, sum_in, max_in, seg, wms, wmd, wmr, c0,
      bm1, wm2t, bm2, gn, bn, wg1t, bg1, wg2t, bg2)


def kernel(block_features, typed_edges, W1, b1, g1, beta1, W2, b2, rel_emb,
           Wm1, bm1, Wm2, bm2, gn, bn, Wg1, bg1, Wg2, bg2):
    te = typed_edges.astype(jnp.int32)
    tw = jnp.array([64.0, 512.0, 8.0, 1.0], dtype=jnp.float32)
    t_idx = jnp.sum(te.astype(jnp.float32) * tw[None, :],
                    axis=1).astype(jnp.int32)
    hists = _sc_hist(t_idx)

    r2 = lambda v: v.reshape(1, _H)
    nodes8, sum_in, max_in = _encode_pool(
        block_features, W1.T, r2(b1), r2(g1), r2(beta1), W2.T, r2(b2),
        r2(gn), r2(bn))

    wms = Wm1[:, :_H].T
    wmd = Wm1[:, _H:2 * _H].T
    wmr = Wm1[:, 2 * _H:3 * _H].T
    c0 = Wm1[:, 3 * _H].reshape(1, _H)

    out = _finalize(hists, nodes8, rel_emb, sum_in, max_in, jnp.asarray(_SEG),
                    wms, wmd, wmr, c0, r2(bm1), Wm2.T, r2(bm2), r2(gn),
                    r2(bn), Wg1.T, r2(bg1), Wg2.T, r2(bg2))
    return out.reshape(_H)


# speedup vs baseline: 1.3711x; 1.1235x over previous
import functools

import numpy as np

import jax
import jax.numpy as jnp
from jax import lax
from jax.experimental import pallas as pl
from jax.experimental.pallas import tpu as pltpu
from jax.experimental.pallas import tpu_sc as plsc

_N = 10000
_D = 128
_H = 64
_R = 7
_E = 320000

_NW = 32
_EPW = _E // _NW
_LANES = 16
_BINS = 4096
_BN = 1000
_NG = _N // _BN

_SEG = np.repeat(np.eye(8, dtype=np.float32), _BINS // 8, axis=1)


def _sc_hist(t_idx):
    mesh = plsc.VectorSubcoreMesh(core_axis_name="c", subcore_axis_name="s")

    @functools.partial(
        pl.kernel,
        mesh=mesh,
        out_type=jax.ShapeDtypeStruct((_NW * _BINS,), jnp.float32),
        scratch_types=[
            pltpu.VMEM((_EPW,), jnp.int32),
            pltpu.VMEM((_BINS * _LANES,), jnp.float32),
            pltpu.VMEM((_BINS,), jnp.float32),
        ],
        compiler_params=pltpu.CompilerParams(needs_layout_passes=False),
    )
    def hist(t_hbm, out_hbm, tv, hv, rv):
        wid = lax.axis_index("s") * 2 + lax.axis_index("c")
        pltpu.sync_copy(t_hbm.at[pl.ds(wid * _EPW, _EPW)], tv)

        zero16 = jnp.zeros((_LANES,), jnp.float32)

        def zbody(i, carry):
            for u in range(8):
                hv[pl.ds(i * 128 + u * 16, 16)] = zero16
            return carry

        lax.fori_loop(0, _BINS * _LANES // 128, zbody, 0)

        lanes = lax.iota(jnp.int32, _LANES)
        lane_off = lanes * _BINS
        ones = jnp.ones((_LANES,), jnp.float32)

        def body(i, carry):
            t = tv[pl.ds(i * _LANES, _LANES)]
            plsc.addupdate_scatter(hv, [t + lane_off], ones)
            return carry

        lax.fori_loop(0, _EPW // _LANES, body, 0)

        def rbody(v, carry):
            acc = hv[pl.ds(v * 16, 16)]
            for l in range(1, _LANES):
                acc = acc + hv[pl.ds(l * _BINS + v * 16, 16)]
            rv[pl.ds(v * 16, 16)] = acc
            return carry

        lax.fori_loop(0, _BINS // 16, rbody, 0)
        pltpu.sync_copy(rv, out_hbm.at[pl.ds(wid * _BINS, _BINS)])

    return hist(t_idx)


def _tdot(x, w):
    return lax.dot_general(x, w, (((1,), (1,)), ((), ())),
                           preferred_element_type=jnp.float32)


def _enc_body(x_ref, w1_ref, b1_ref, g1_ref, beta1_ref, w2_ref, b2_ref,
              gn_ref, bn_ref, nodes8_ref, sum_ref, max_ref):
    g = pl.program_id(0)
    x = x_ref[...]
    h = _tdot(x, w1_ref[...]) + b1_ref[...]
    mu = jnp.mean(h, axis=-1, keepdims=True)
    var = jnp.mean((h - mu) ** 2, axis=-1, keepdims=True)
    h = (h - mu) / jnp.sqrt(var + 1e-5) * g1_ref[...] + beta1_ref[...]
    h = jnp.maximum(h, 0.0)
    n = _tdot(h, w2_ref[...])
    n = jnp.maximum(n + b2_ref[...], 0.0)

    @pl.when(g == 0)
    def _():
        nodes8_ref[...] = n[:8]
        sum_ref[...] = jnp.zeros((1, _H), jnp.float32)
        max_ref[...] = jnp.full((1, _H), -jnp.inf, jnp.float32)

    mu2 = jnp.mean(n, axis=-1, keepdims=True)
    var2 = jnp.mean((n - mu2) ** 2, axis=-1, keepdims=True)
    ln = (n - mu2) / jnp.sqrt(var2 + 1e-5) * gn_ref[...] + bn_ref[...]
    rows = lax.broadcasted_iota(jnp.int32, (_BN, 1), 0) + g * _BN
    keep = rows >= 8
    sum_ref[...] += jnp.sum(jnp.where(keep, ln, 0.0), axis=0, keepdims=True)
    max_ref[...] = jnp.maximum(
        max_ref[...],
        jnp.max(jnp.where(keep, ln, -jnp.inf), axis=0, keepdims=True))


def _encode_pool(block_features, w1, b1, g1, beta1, w2, b2, gn, bn):
    full = lambda s: pl.BlockSpec(s, lambda i: (0, 0))
    return pl.pallas_call(
        _enc_body,
        grid=(_NG,),
        in_specs=[
            pl.BlockSpec((_BN, _D), lambda i: (i, 0)),
            full((_H, _D)), full((1, _H)), full((1, _H)), full((1, _H)),
            full((_H, _H)), full((1, _H)), full((1, _H)), full((1, _H)),
        ],
        out_specs=[full((8, _H)), full((1, _H)), full((1, _H))],
        out_shape=[
            jax.ShapeDtypeStruct((8, _H), jnp.float32),
            jax.ShapeDtypeStruct((1, _H), jnp.float32),
            jax.ShapeDtypeStruct((1, _H), jnp.float32),
        ],
    )(block_features, w1, b1, g1, beta1, w2, b2, gn, bn)


def _fin_body(hists_ref, nodes8_ref, rel_ref, sumin_ref, maxin_ref, seg_ref,
              wm1_ref, bm1_ref, wm2_ref, bm2_ref,
              gn_ref, bn_ref, wg1_ref, bg1_ref, wg2_ref, bg2_ref, out_ref):
    counts = jnp.sum(hists_ref[...].reshape(_NW, _BINS), axis=0)
    n8 = nodes8_ref[...]
    rel8 = jnp.concatenate(
        [rel_ref[...], jnp.zeros((1, _H), jnp.float32)], axis=0)
    wm1 = wm1_ref[...]
    pre_s = _tdot(n8, wm1[:, :_H])
    pre_d = _tdot(n8, wm1[:, _H:2 * _H])
    pre_r = _tdot(rel8, wm1[:, 2 * _H:3 * _H])
    wvals = lax.broadcasted_iota(jnp.int32, (8, 1), 0).astype(jnp.float32)
    pre_w = _tdot(wvals, wm1[:, 3 * _H:])
    ds = (pre_d[:, None, :] + pre_s[None, :, :]).reshape(64, _H)
    rw = (pre_r[:, None, :] + pre_w[None, :, :]).reshape(64, _H)
    h1 = jnp.maximum(ds[:, None, :] + rw[None, :, :] + bm1_ref[...], 0.0)
    h1 = h1.reshape(_BINS, _H)
    msg = _tdot(h1, wm2_ref[...]) + bm2_ref[...]
    wmsg = msg * counts[:, None]
    agg8 = jnp.dot(seg_ref[...], wmsg,
                   preferred_element_type=jnp.float32)
    x8 = n8 + agg8
    mu = jnp.mean(x8, axis=-1, keepdims=True)
    var = jnp.mean((x8 - mu) ** 2, axis=-1, keepdims=True)
    ln8 = (x8 - mu) / jnp.sqrt(var + 1e-5) * gn_ref[...] + bn_ref[...]
    total = sumin_ref[...] + jnp.sum(ln8, axis=0, keepdims=True)
    mx = jnp.maximum(maxin_ref[...], jnp.max(ln8, axis=0, keepdims=True))
    graph = jnp.concatenate([total / float(_N), mx], axis=-1)
    z = jnp.maximum(_tdot(graph, wg1_ref[...]) + bg1_ref[...], 0.0)
    o = _tdot(z, wg2_ref[...])
    out_ref[...] = o + bg2_ref[...]


def _finalize(hists, nodes8, rel_emb, sum_in, max_in, seg, wm1, bm1, wm2,
              bm2, gn, bn, wg1, bg1, wg2, bg2):
    return pl.pallas_call(
        _fin_body,
        out_shape=jax.ShapeDtypeStruct((1, _H), jnp.float32),
    )(hists, nodes8, rel_emb, sum_in, max_in, seg, wm1, bm1, wm2, bm2,
      gn, bn, wg1, bg1, wg2, bg2)


def kernel(block_features, typed_edges, W1, b1, g1, beta1, W2, b2, rel_emb,
           Wm1, bm1, Wm2, bm2, gn, bn, Wg1, bg1, Wg2, bg2):
    te = typed_edges.astype(jnp.int32)
    tw = jnp.array([64, 512, 8, 1], dtype=jnp.int32)
    t_idx = jnp.sum(te * tw[None, :], axis=1)
    hists = _sc_hist(t_idx)

    r2 = lambda v: v.reshape(1, _H)
    nodes8, sum_in, max_in = _encode_pool(
        block_features, W1, r2(b1), r2(g1), r2(beta1), W2, r2(b2),
        r2(gn), r2(bn))

    out = _finalize(hists, nodes8, rel_emb, sum_in, max_in, jnp.asarray(_SEG),
                    Wm1, r2(bm1), Wm2, r2(bm2), r2(gn), r2(bn),
                    Wg1, r2(bg1), Wg2, r2(bg2))
    return out.reshape(_H)


# speedup vs baseline: 1.3715x; 1.0004x over previous
import functools

import numpy as np

import jax
import jax.numpy as jnp
from jax import lax
from jax.experimental import pallas as pl
from jax.experimental.pallas import tpu as pltpu
from jax.experimental.pallas import tpu_sc as plsc

_N = 10000
_D = 128
_H = 64
_R = 7
_E = 320000

_NW = 32
_EPW = _E // _NW
_LANES = 16
_BINS = 4096
_BN = 1000
_NG = _N // _BN

_SEG = np.repeat(np.eye(8, dtype=np.float32), _BINS // 8, axis=1)


def _sc_hist(t_idx):
    mesh = plsc.VectorSubcoreMesh(core_axis_name="c", subcore_axis_name="s")

    @functools.partial(
        pl.kernel,
        mesh=mesh,
        out_type=jax.ShapeDtypeStruct((_NW * _BINS,), jnp.float32),
        scratch_types=[
            pltpu.VMEM((_EPW,), jnp.int32),
            pltpu.VMEM((_BINS * _LANES,), jnp.float32),
            pltpu.VMEM((_BINS,), jnp.float32),
        ],
        compiler_params=pltpu.CompilerParams(needs_layout_passes=False),
    )
    def hist(t_hbm, out_hbm, tv, hv, rv):
        wid = lax.axis_index("s") * 2 + lax.axis_index("c")
        pltpu.sync_copy(t_hbm.at[pl.ds(wid * _EPW, _EPW)], tv)

        zero16 = jnp.zeros((_LANES,), jnp.float32)

        def zbody(i, carry):
            for u in range(8):
                hv[pl.ds(i * 128 + u * 16, 16)] = zero16
            return carry

        lax.fori_loop(0, _BINS * _LANES // 128, zbody, 0)

        lanes = lax.iota(jnp.int32, _LANES)
        lane_off = lanes * _BINS
        ones = jnp.ones((_LANES,), jnp.float32)

        def body(i, carry):
            t = tv[pl.ds(i * _LANES, _LANES)]
            plsc.addupdate_scatter(hv, [t + lane_off], ones)
            return carry

        lax.fori_loop(0, _EPW // _LANES, body, 0)

        def rbody(v, carry):
            acc = hv[pl.ds(v * 16, 16)]
            for l in range(1, _LANES):
                acc = acc + hv[pl.ds(l * _BINS + v * 16, 16)]
            rv[pl.ds(v * 16, 16)] = acc
            return carry

        lax.fori_loop(0, _BINS // 16, rbody, 0)
        pltpu.sync_copy(rv, out_hbm.at[pl.ds(wid * _BINS, _BINS)])

    return hist(t_idx)


def _tdot(x, w):
    return lax.dot_general(x, w, (((1,), (1,)), ((), ())),
                           preferred_element_type=jnp.float32)


def _enc_body(x_ref, w1_ref, b1_ref, g1_ref, beta1_ref, w2_ref, b2_ref,
              gn_ref, bn_ref, nodes8_ref, sum_ref, max_ref):
    g = pl.program_id(0)
    x = x_ref[...]
    h = _tdot(x, w1_ref[...]) + b1_ref[...]
    mu = jnp.mean(h, axis=-1, keepdims=True)
    var = jnp.mean((h - mu) ** 2, axis=-1, keepdims=True)
    h = (h - mu) / jnp.sqrt(var + 1e-5) * g1_ref[...] + beta1_ref[...]
    h = jnp.maximum(h, 0.0)
    n = _tdot(h, w2_ref[...])
    n = jnp.maximum(n + b2_ref[...], 0.0)

    mu2 = jnp.mean(n, axis=-1, keepdims=True)
    var2 = jnp.mean((n - mu2) ** 2, axis=-1, keepdims=True)
    ln = (n - mu2) / jnp.sqrt(var2 + 1e-5) * gn_ref[...] + bn_ref[...]

    @pl.when(g == 0)
    def _():
        nodes8_ref[...] = n[:8]
        keep = lax.broadcasted_iota(jnp.int32, (_BN, 1), 0) >= 8
        sum_ref[...] = jnp.sum(jnp.where(keep, ln, 0.0), axis=0,
                               keepdims=True)
        max_ref[...] = jnp.max(jnp.where(keep, ln, -jnp.inf), axis=0,
                               keepdims=True)

    @pl.when(g > 0)
    def _():
        sum_ref[...] += jnp.sum(ln, axis=0, keepdims=True)
        max_ref[...] = jnp.maximum(max_ref[...],
                                   jnp.max(ln, axis=0, keepdims=True))


def _encode_pool(block_features, w1, b1, g1, beta1, w2, b2, gn, bn):
    full = lambda s: pl.BlockSpec(s, lambda i: (0, 0))
    return pl.pallas_call(
        _enc_body,
        grid=(_NG,),
        in_specs=[
            pl.BlockSpec((_BN, _D), lambda i: (i, 0)),
            full((_H, _D)), full((1, _H)), full((1, _H)), full((1, _H)),
            full((_H, _H)), full((1, _H)), full((1, _H)), full((1, _H)),
        ],
        out_specs=[full((8, _H)), full((1, _H)), full((1, _H))],
        out_shape=[
            jax.ShapeDtypeStruct((8, _H), jnp.float32),
            jax.ShapeDtypeStruct((1, _H), jnp.float32),
            jax.ShapeDtypeStruct((1, _H), jnp.float32),
        ],
    )(block_features, w1, b1, g1, beta1, w2, b2, gn, bn)


def _fin_body(hists_ref, nodes8_ref, rel_ref, sumin_ref, maxin_ref, seg_ref,
              wm1_ref, bm1_ref, wm2_ref, bm2_ref,
              gn_ref, bn_ref, wg1_ref, bg1_ref, wg2_ref, bg2_ref, out_ref):
    counts = jnp.sum(hists_ref[...].reshape(_NW, _BINS), axis=0)
    n8 = nodes8_ref[...]
    rel8 = jnp.concatenate(
        [rel_ref[...], jnp.zeros((1, _H), jnp.float32)], axis=0)
    wm1 = wm1_ref[...]
    pre_s = _tdot(n8, wm1[:, :_H])
    pre_d = _tdot(n8, wm1[:, _H:2 * _H])
    pre_r = _tdot(rel8, wm1[:, 2 * _H:3 * _H])
    wvals = lax.broadcasted_iota(jnp.int32, (8, 1), 0).astype(jnp.float32)
    pre_w = _tdot(wvals, wm1[:, 3 * _H:])
    ds = (pre_d[:, None, :] + pre_s[None, :, :]).reshape(64, _H)
    rw = (pre_r[:, None, :] + pre_w[None, :, :]).reshape(64, _H)
    h1 = jnp.maximum(ds[:, None, :] + rw[None, :, :] + bm1_ref[...], 0.0)
    h1 = h1.reshape(_BINS, _H)
    msg = _tdot(h1, wm2_ref[...]) + bm2_ref[...]
    wmsg = msg * counts[:, None]
    agg8 = jnp.dot(seg_ref[...], wmsg,
                   preferred_element_type=jnp.float32)
    x8 = n8 + agg8
    mu = jnp.mean(x8, axis=-1, keepdims=True)
    var = jnp.mean((x8 - mu) ** 2, axis=-1, keepdims=True)
    ln8 = (x8 - mu) / jnp.sqrt(var + 1e-5) * gn_ref[...] + bn_ref[...]
    total = sumin_ref[...] + jnp.sum(ln8, axis=0, keepdims=True)
    mx = jnp.maximum(maxin_ref[...], jnp.max(ln8, axis=0, keepdims=True))
    graph = jnp.concatenate([total / float(_N), mx], axis=-1)
    z = jnp.maximum(_tdot(graph, wg1_ref[...]) + bg1_ref[...], 0.0)
    o = _tdot(z, wg2_ref[...])
    out_ref[...] = o + bg2_ref[...]


def _finalize(hists, nodes8, rel_emb, sum_in, max_in, seg, wm1, bm1, wm2,
              bm2, gn, bn, wg1, bg1, wg2, bg2):
    return pl.pallas_call(
        _fin_body,
        out_shape=jax.ShapeDtypeStruct((1, _H), jnp.float32),
    )(hists, nodes8, rel_emb, sum_in, max_in, seg, wm1, bm1, wm2, bm2,
      gn, bn, wg1, bg1, wg2, bg2)


def kernel(block_features, typed_edges, W1, b1, g1, beta1, W2, b2, rel_emb,
           Wm1, bm1, Wm2, bm2, gn, bn, Wg1, bg1, Wg2, bg2):
    te = typed_edges.astype(jnp.int32)
    tw = jnp.array([64, 512, 8, 1], dtype=jnp.int32)
    t_idx = jnp.sum(te * tw[None, :], axis=1)
    hists = _sc_hist(t_idx)

    r2 = lambda v: v.reshape(1, _H)
    nodes8, sum_in, max_in = _encode_pool(
        block_features, W1, r2(b1), r2(g1), r2(beta1), W2, r2(b2),
        r2(gn), r2(bn))

    out = _finalize(hists, nodes8, rel_emb, sum_in, max_in, jnp.asarray(_SEG),
                    Wm1, r2(bm1), Wm2, r2(bm2), r2(gn), r2(bn),
                    Wg1, r2(bg1), Wg2, r2(bg2))
    return out.reshape(_H)


# speedup vs baseline: 1.6470x; 1.2008x over previous
import functools

import numpy as np

import jax
import jax.numpy as jnp
from jax import lax
from jax.experimental import pallas as pl
from jax.experimental.pallas import tpu as pltpu
from jax.experimental.pallas import tpu_sc as plsc

_N = 10000
_D = 128
_H = 64
_R = 7
_E = 320000

_NW = 32
_EPW = _E // _NW
_LANES = 16
_BINS = 4096
_BN = 1000
_NG = _N // _BN

_SEG = np.repeat(np.eye(8, dtype=np.float32), _BINS // 8, axis=1)


_BLK = 128
_NBLK = _E // _BLK
_BPW = _NBLK // _NW
_XTRA = _NBLK - _BPW * _NW
_CHB = _BPW // 2
_CH = _CHB * _BLK


def _sc_hist(te_t):
    mesh = plsc.VectorSubcoreMesh(core_axis_name="c", subcore_axis_name="s")

    @functools.partial(
        pl.kernel,
        mesh=mesh,
        out_type=jax.ShapeDtypeStruct((_NW * _BINS,), jnp.float32),
        scratch_types=[
            pltpu.VMEM((4, _CH), jnp.int32),
            pltpu.VMEM((4, _BLK), jnp.int32),
            pltpu.VMEM((_BINS * _LANES,), jnp.float32),
            pltpu.VMEM((_BINS,), jnp.float32),
        ],
        compiler_params=pltpu.CompilerParams(needs_layout_passes=False),
    )
    def hist(te_hbm, out_hbm, ev, xv, hv, rv):
        wid = lax.axis_index("s") * 2 + lax.axis_index("c")
        col0 = (_BPW * wid + jnp.minimum(wid, _XTRA)) * _BLK

        zero16 = jnp.zeros((_LANES,), jnp.float32)

        def zbody(i, carry):
            for u in range(8):
                hv[pl.ds(i * 128 + u * 16, 16)] = zero16
            return carry

        lax.fori_loop(0, _BINS * _LANES // 128, zbody, 0)

        lanes = lax.iota(jnp.int32, _LANES)
        lane_off = lanes * _BINS
        ones = jnp.ones((_LANES,), jnp.float32)

        def acc16(src, o):
            s = src[0, pl.ds(o, _LANES)]
            d = src[1, pl.ds(o, _LANES)]
            r = src[2, pl.ds(o, _LANES)]
            w = src[3, pl.ds(o, _LANES)]
            t = ((d * 8 + s) * 8 + r) * 8 + w
            plsc.addupdate_scatter(hv, [t + lane_off], ones)

        def body(i, carry):
            acc16(ev, i * _LANES)
            return carry

        for c in range(2):
            pltpu.sync_copy(te_hbm.at[:, pl.ds(col0 + c * _CH, _CH)], ev)
            lax.fori_loop(0, _CH // _LANES, body, 0)

        @pl.when(wid < _XTRA)
        def _():
            pltpu.sync_copy(te_hbm.at[:, pl.ds(col0 + 2 * _CH, _BLK)], xv)

            def xbody(i, carry):
                acc16(xv, i * _LANES)
                return carry

            lax.fori_loop(0, _BLK // _LANES, xbody, 0)

        def rbody(v, carry):
            acc = hv[pl.ds(v * 16, 16)]
            for l in range(1, _LANES):
                acc = acc + hv[pl.ds(l * _BINS + v * 16, 16)]
            rv[pl.ds(v * 16, 16)] = acc
            return carry

        lax.fori_loop(0, _BINS // 16, rbody, 0)
        pltpu.sync_copy(rv, out_hbm.at[pl.ds(wid * _BINS, _BINS)])

    return hist(te_t)


def _tdot(x, w):
    return lax.dot_general(x, w, (((1,), (1,)), ((), ())),
                           preferred_element_type=jnp.float32)


def _enc_body(x_ref, w1_ref, b1_ref, g1_ref, beta1_ref, w2_ref, b2_ref,
              gn_ref, bn_ref, nodes8_ref, sum_ref, max_ref):
    g = pl.program_id(0)
    x = x_ref[...]
    h = _tdot(x, w1_ref[...]) + b1_ref[...]
    mu = jnp.mean(h, axis=-1, keepdims=True)
    var = jnp.mean((h - mu) ** 2, axis=-1, keepdims=True)
    h = (h - mu) / jnp.sqrt(var + 1e-5) * g1_ref[...] + beta1_ref[...]
    h = jnp.maximum(h, 0.0)
    n = _tdot(h, w2_ref[...])
    n = jnp.maximum(n + b2_ref[...], 0.0)

    mu2 = jnp.mean(n, axis=-1, keepdims=True)
    var2 = jnp.mean((n - mu2) ** 2, axis=-1, keepdims=True)
    ln = (n - mu2) / jnp.sqrt(var2 + 1e-5) * gn_ref[...] + bn_ref[...]

    @pl.when(g == 0)
    def _():
        nodes8_ref[...] = n[:8]
        keep = lax.broadcasted_iota(jnp.int32, (_BN, 1), 0) >= 8
        sum_ref[...] = jnp.sum(jnp.where(keep, ln, 0.0), axis=0,
                               keepdims=True)
        max_ref[...] = jnp.max(jnp.where(keep, ln, -jnp.inf), axis=0,
                               keepdims=True)

    @pl.when(g > 0)
    def _():
        sum_ref[...] += jnp.sum(ln, axis=0, keepdims=True)
        max_ref[...] = jnp.maximum(max_ref[...],
                                   jnp.max(ln, axis=0, keepdims=True))


def _encode_pool(block_features, w1, b1, g1, beta1, w2, b2, gn, bn):
    full = lambda s: pl.BlockSpec(s, lambda i: (0, 0))
    return pl.pallas_call(
        _enc_body,
        grid=(_NG,),
        in_specs=[
            pl.BlockSpec((_BN, _D), lambda i: (i, 0)),
            full((_H, _D)), full((1, _H)), full((1, _H)), full((1, _H)),
            full((_H, _H)), full((1, _H)), full((1, _H)), full((1, _H)),
        ],
        out_specs=[full((8, _H)), full((1, _H)), full((1, _H))],
        out_shape=[
            jax.ShapeDtypeStruct((8, _H), jnp.float32),
            jax.ShapeDtypeStruct((1, _H), jnp.float32),
            jax.ShapeDtypeStruct((1, _H), jnp.float32),
        ],
    )(block_features, w1, b1, g1, beta1, w2, b2, gn, bn)


def _fin_body(hists_ref, nodes8_ref, rel_ref, sumin_ref, maxin_ref, seg_ref,
              wm1_ref, bm1_ref, wm2_ref, bm2_ref,
              gn_ref, bn_ref, wg1_ref, bg1_ref, wg2_ref, bg2_ref, out_ref):
    counts = jnp.sum(hists_ref[...].reshape(_NW, _BINS), axis=0)
    n8 = nodes8_ref[...]
    rel8 = jnp.concatenate(
        [rel_ref[...], jnp.zeros((1, _H), jnp.float32)], axis=0)
    wm1 = wm1_ref[...]
    pre_s = _tdot(n8, wm1[:, :_H])
    pre_d = _tdot(n8, wm1[:, _H:2 * _H])
    pre_r = _tdot(rel8, wm1[:, 2 * _H:3 * _H])
    wvals = lax.broadcasted_iota(jnp.int32, (8, 1), 0).astype(jnp.float32)
    pre_w = _tdot(wvals, wm1[:, 3 * _H:])
    ds = (pre_d[:, None, :] + pre_s[None, :, :]).reshape(64, _H)
    rw = (pre_r[:, None, :] + pre_w[None, :, :]).reshape(64, _H)
    h1 = jnp.maximum(ds[:, None, :] + rw[None, :, :] + bm1_ref[...], 0.0)
    h1 = h1.reshape(_BINS, _H)
    msg = _tdot(h1, wm2_ref[...]) + bm2_ref[...]
    wmsg = msg * counts[:, None]
    agg8 = jnp.dot(seg_ref[...], wmsg,
                   preferred_element_type=jnp.float32)
    x8 = n8 + agg8
    mu = jnp.mean(x8, axis=-1, keepdims=True)
    var = jnp.mean((x8 - mu) ** 2, axis=-1, keepdims=True)
    ln8 = (x8 - mu) / jnp.sqrt(var + 1e-5) * gn_ref[...] + bn_ref[...]
    total = sumin_ref[...] + jnp.sum(ln8, axis=0, keepdims=True)
    mx = jnp.maximum(maxin_ref[...], jnp.max(ln8, axis=0, keepdims=True))
    graph = jnp.concatenate([total / float(_N), mx], axis=-1)
    z = jnp.maximum(_tdot(graph, wg1_ref[...]) + bg1_ref[...], 0.0)
    o = _tdot(z, wg2_ref[...])
    out_ref[...] = o + bg2_ref[...]


def _finalize(hists, nodes8, rel_emb, sum_in, max_in, seg, wm1, bm1, wm2,
              bm2, gn, bn, wg1, bg1, wg2, bg2):
    return pl.pallas_call(
        _fin_body,
        out_shape=jax.ShapeDtypeStruct((1, _H), jnp.float32),
    )(hists, nodes8, rel_emb, sum_in, max_in, seg, wm1, bm1, wm2, bm2,
      gn, bn, wg1, bg1, wg2, bg2)


def kernel(block_features, typed_edges, W1, b1, g1, beta1, W2, b2, rel_emb,
           Wm1, bm1, Wm2, bm2, gn, bn, Wg1, bg1, Wg2, bg2):
    hists = _sc_hist(typed_edges.astype(jnp.int32).T)

    r2 = lambda v: v.reshape(1, _H)
    nodes8, sum_in, max_in = _encode_pool(
        block_features, W1, r2(b1), r2(g1), r2(beta1), W2, r2(b2),
        r2(gn), r2(bn))

    out = _finalize(hists, nodes8, rel_emb, sum_in, max_in, jnp.asarray(_SEG),
                    Wm1, r2(bm1), Wm2, r2(bm2), r2(gn), r2(bn),
                    Wg1, r2(bg1), Wg2, r2(bg2))
    return out.reshape(_H)


# speedup vs baseline: 2.0504x; 1.2450x over previous
import functools

import numpy as np

import jax
import jax.numpy as jnp
from jax import lax
from jax.experimental import pallas as pl
from jax.experimental.pallas import tpu as pltpu
from jax.experimental.pallas import tpu_sc as plsc

_N = 10000
_D = 128
_H = 64
_R = 7
_E = 320000

_NW = 32
_EPW = _E // _NW
_LANES = 16
_BINS = 4096
_BN = 1000
_NG = _N // _BN

_SEG = np.repeat(np.eye(8, dtype=np.float32), _BINS // 8, axis=1)


_BLK = 128
_NBLK = _E // _BLK
_BPW = _NBLK // _NW
_XTRA = _NBLK - _BPW * _NW
_CHB = _BPW // 2
_CH = _CHB * _BLK


def _sc_hist(te_t):
    mesh = plsc.VectorSubcoreMesh(core_axis_name="c", subcore_axis_name="s")

    @functools.partial(
        pl.kernel,
        mesh=mesh,
        out_type=jax.ShapeDtypeStruct((_NW * _BINS,), jnp.float32),
        scratch_types=[
            pltpu.VMEM((4, _CH), jnp.int32),
            pltpu.VMEM((4, _BLK), jnp.int32),
            pltpu.VMEM((_BINS,), jnp.float32),
        ],
        compiler_params=pltpu.CompilerParams(needs_layout_passes=False),
    )
    def hist(te_hbm, out_hbm, ev, xv, hv):
        wid = lax.axis_index("s") * 2 + lax.axis_index("c")
        col0 = (_BPW * wid + jnp.minimum(wid, _XTRA)) * _BLK

        zero16 = jnp.zeros((_LANES,), jnp.float32)

        def zbody(i, carry):
            for u in range(8):
                hv[pl.ds(i * 128 + u * 16, 16)] = zero16
            return carry

        lax.fori_loop(0, _BINS // 128, zbody, 0)

        ones = jnp.ones((_LANES,), jnp.float32)

        def acc16(src, o):
            s = src[0, pl.ds(o, _LANES)]
            d = src[1, pl.ds(o, _LANES)]
            r = src[2, pl.ds(o, _LANES)]
            w = src[3, pl.ds(o, _LANES)]
            t = ((d * 8 + s) * 8 + r) * 8 + w
            plsc.addupdate_scatter(hv, [t], ones)

        def body(i, carry):
            acc16(ev, i * _LANES)
            return carry

        for c in range(2):
            pltpu.sync_copy(te_hbm.at[:, pl.ds(col0 + c * _CH, _CH)], ev)
            lax.fori_loop(0, _CH // _LANES, body, 0)

        @pl.when(wid < _XTRA)
        def _():
            pltpu.sync_copy(te_hbm.at[:, pl.ds(col0 + 2 * _CH, _BLK)], xv)

            def xbody(i, carry):
                acc16(xv, i * _LANES)
                return carry

            lax.fori_loop(0, _BLK // _LANES, xbody, 0)

        pltpu.sync_copy(hv, out_hbm.at[pl.ds(wid * _BINS, _BINS)])

    return hist(te_t)


def _tdot(x, w):
    return lax.dot_general(x, w, (((1,), (1,)), ((), ())),
                           preferred_element_type=jnp.float32)


def _enc_body(x_ref, w1_ref, b1_ref, g1_ref, beta1_ref, w2_ref, b2_ref,
              gn_ref, bn_ref, nodes8_ref, sum_ref, max_ref):
    g = pl.program_id(0)
    x = x_ref[...]
    h = _tdot(x, w1_ref[...]) + b1_ref[...]
    mu = jnp.mean(h, axis=-1, keepdims=True)
    var = jnp.mean((h - mu) ** 2, axis=-1, keepdims=True)
    h = (h - mu) / jnp.sqrt(var + 1e-5) * g1_ref[...] + beta1_ref[...]
    h = jnp.maximum(h, 0.0)
    n = _tdot(h, w2_ref[...])
    n = jnp.maximum(n + b2_ref[...], 0.0)

    mu2 = jnp.mean(n, axis=-1, keepdims=True)
    var2 = jnp.mean((n - mu2) ** 2, axis=-1, keepdims=True)
    ln = (n - mu2) / jnp.sqrt(var2 + 1e-5) * gn_ref[...] + bn_ref[...]

    @pl.when(g == 0)
    def _():
        nodes8_ref[...] = n[:8]
        keep = lax.broadcasted_iota(jnp.int32, (_BN, 1), 0) >= 8
        sum_ref[...] = jnp.sum(jnp.where(keep, ln, 0.0), axis=0,
                               keepdims=True)
        max_ref[...] = jnp.max(jnp.where(keep, ln, -jnp.inf), axis=0,
                               keepdims=True)

    @pl.when(g > 0)
    def _():
        sum_ref[...] += jnp.sum(ln, axis=0, keepdims=True)
        max_ref[...] = jnp.maximum(max_ref[...],
                                   jnp.max(ln, axis=0, keepdims=True))


def _encode_pool(block_features, w1, b1, g1, beta1, w2, b2, gn, bn):
    full = lambda s: pl.BlockSpec(s, lambda i: (0, 0))
    return pl.pallas_call(
        _enc_body,
        grid=(_NG,),
        in_specs=[
            pl.BlockSpec((_BN, _D), lambda i: (i, 0)),
            full((_H, _D)), full((1, _H)), full((1, _H)), full((1, _H)),
            full((_H, _H)), full((1, _H)), full((1, _H)), full((1, _H)),
        ],
        out_specs=[full((8, _H)), full((1, _H)), full((1, _H))],
        out_shape=[
            jax.ShapeDtypeStruct((8, _H), jnp.float32),
            jax.ShapeDtypeStruct((1, _H), jnp.float32),
            jax.ShapeDtypeStruct((1, _H), jnp.float32),
        ],
    )(block_features, w1, b1, g1, beta1, w2, b2, gn, bn)


def _fin_body(hists_ref, nodes8_ref, rel_ref, sumin_ref, maxin_ref, seg_ref,
              wm1_ref, bm1_ref, wm2_ref, bm2_ref,
              gn_ref, bn_ref, wg1_ref, bg1_ref, wg2_ref, bg2_ref, out_ref):
    counts = jnp.sum(hists_ref[...].reshape(_NW, _BINS), axis=0)
    n8 = nodes8_ref[...]
    rel8 = jnp.concatenate(
        [rel_ref[...], jnp.zeros((1, _H), jnp.float32)], axis=0)
    wm1 = wm1_ref[...]
    pre_s = _tdot(n8, wm1[:, :_H])
    pre_d = _tdot(n8, wm1[:, _H:2 * _H])
    pre_r = _tdot(rel8, wm1[:, 2 * _H:3 * _H])
    wvals = lax.broadcasted_iota(jnp.int32, (8, 1), 0).astype(jnp.float32)
    pre_w = _tdot(wvals, wm1[:, 3 * _H:])
    ds = (pre_d[:, None, :] + pre_s[None, :, :]).reshape(64, _H)
    rw = (pre_r[:, None, :] + pre_w[None, :, :]).reshape(64, _H)
    h1 = jnp.maximum(ds[:, None, :] + rw[None, :, :] + bm1_ref[...], 0.0)
    h1 = h1.reshape(_BINS, _H)
    msg = _tdot(h1, wm2_ref[...]) + bm2_ref[...]
    wmsg = msg * counts[:, None]
    agg8 = jnp.dot(seg_ref[...], wmsg,
                   preferred_element_type=jnp.float32)
    x8 = n8 + agg8
    mu = jnp.mean(x8, axis=-1, keepdims=True)
    var = jnp.mean((x8 - mu) ** 2, axis=-1, keepdims=True)
    ln8 = (x8 - mu) / jnp.sqrt(var + 1e-5) * gn_ref[...] + bn_ref[...]
    total = sumin_ref[...] + jnp.sum(ln8, axis=0, keepdims=True)
    mx = jnp.maximum(maxin_ref[...], jnp.max(ln8, axis=0, keepdims=True))
    graph = jnp.concatenate([total / float(_N), mx], axis=-1)
    z = jnp.maximum(_tdot(graph, wg1_ref[...]) + bg1_ref[...], 0.0)
    o = _tdot(z, wg2_ref[...])
    out_ref[...] = o + bg2_ref[...]


def _finalize(hists, nodes8, rel_emb, sum_in, max_in, seg, wm1, bm1, wm2,
              bm2, gn, bn, wg1, bg1, wg2, bg2):
    return pl.pallas_call(
        _fin_body,
        out_shape=jax.ShapeDtypeStruct((1, _H), jnp.float32),
    )(hists, nodes8, rel_emb, sum_in, max_in, seg, wm1, bm1, wm2, bm2,
      gn, bn, wg1, bg1, wg2, bg2)


def kernel(block_features, typed_edges, W1, b1, g1, beta1, W2, b2, rel_emb,
           Wm1, bm1, Wm2, bm2, gn, bn, Wg1, bg1, Wg2, bg2):
    hists = _sc_hist(typed_edges.astype(jnp.int32).T)

    r2 = lambda v: v.reshape(1, _H)
    nodes8, sum_in, max_in = _encode_pool(
        block_features, W1, r2(b1), r2(g1), r2(beta1), W2, r2(b2),
        r2(gn), r2(bn))

    out = _finalize(hists, nodes8, rel_emb, sum_in, max_in, jnp.asarray(_SEG),
                    Wm1, r2(bm1), Wm2, r2(bm2), r2(gn), r2(bn),
                    Wg1, r2(bg1), Wg2, r2(bg2))
    return out.reshape(_H)
